# baseline (speedup 1.0000x reference)
import functools

import jax
import jax.numpy as jnp
from jax import lax
from jax.experimental import pallas as pl
from jax.experimental.pallas import tpu as pltpu
from jax.experimental.pallas import tpu_sc as plsc

B = 16384
EMB = 64
NUM_LABELS = 5
HID = 200

_NC = 2
_NS = 16
_NW = _NC * _NS
_B_PER_W = B // _NW


@functools.cache
def _make_sc_gather():
    mesh = plsc.VectorSubcoreMesh(core_axis_name="c", subcore_axis_name="s")

    @functools.partial(
        pl.kernel,
        mesh=mesh,
        out_type=jax.ShapeDtypeStruct((B, EMB), jnp.float32),
        scratch_types=[
            pltpu.VMEM((_B_PER_W,), jnp.int32),
            pltpu.VMEM((_B_PER_W, EMB), jnp.float32),
            pltpu.SemaphoreType.DMA,
        ],
        compiler_params=pltpu.CompilerParams(use_tc_tiling_on_sc=False),
    )
    def sc_gather(table_hbm, idx_hbm, out_hbm, idx_v, rows_v, sem):
        wid = lax.axis_index("s") * _NC + lax.axis_index("c")
        base = wid * _B_PER_W
        pltpu.sync_copy(idx_hbm.at[pl.ds(base, _B_PER_W)], idx_v)
        pltpu.async_copy(table_hbm.at[idx_v], rows_v, sem).wait()
        pltpu.sync_copy(rows_v, out_hbm.at[pl.ds(base, _B_PER_W)])

    return sc_gather


def _mlp_body(x1_ref, prev_ref, wh1_ref, wh2_ref, bh_ref, wo_ref, bo_ref, out_ref):
    h = jnp.dot(x1_ref[...], wh1_ref[...], preferred_element_type=jnp.float32)
    h = h + jnp.dot(prev_ref[...], wh2_ref[...], preferred_element_type=jnp.float32)
    h = h + bh_ref[...]
    y = h * jax.nn.sigmoid(h)
    logits = jnp.dot(y, wo_ref[...], preferred_element_type=jnp.float32) + bo_ref[...]
    m = jnp.max(logits, axis=-1, keepdims=True)
    e = jnp.exp(logits - m)
    out_ref[...] = e / jnp.sum(e, axis=-1, keepdims=True)


_BB = 2048


def _mlp(x1, prev, wh1, wh2, bh, wo, bo):
    grid = (B // _BB,)
    return pl.pallas_call(
        _mlp_body,
        grid=grid,
        in_specs=[
            pl.BlockSpec((_BB, EMB), lambda i: (i, 0)),
            pl.BlockSpec((_BB, NUM_LABELS), lambda i: (i, 0)),
            pl.BlockSpec((EMB, HID), lambda i: (0, 0)),
            pl.BlockSpec((NUM_LABELS, HID), lambda i: (0, 0)),
            pl.BlockSpec((1, HID), lambda i: (0, 0)),
            pl.BlockSpec((HID, NUM_LABELS), lambda i: (0, 0)),
            pl.BlockSpec((1, NUM_LABELS), lambda i: (0, 0)),
        ],
        out_specs=pl.BlockSpec((_BB, NUM_LABELS), lambda i: (i, 0)),
        out_shape=jax.ShapeDtypeStruct((B, NUM_LABELS), jnp.float32),
    )(x1, prev, wh1, wh2, bh, wo, bo)


@jax.jit
def kernel(word, prev_label, emb_table, W_h, b_h, W_o, b_o):
    x1 = _make_sc_gather()(emb_table, word)
    return _mlp(
        x1,
        prev_label,
        W_h[:EMB],
        W_h[EMB:],
        b_h.reshape(1, HID),
        W_o,
        b_o.reshape(1, NUM_LABELS),
    )
